# baseline (speedup 1.0000x reference)
import functools

import jax
import jax.numpy as jnp
from jax.experimental import pallas as pl
from jax.experimental.pallas import tpu as pltpu

_EPS = 1e-8
_THRESHOLD = 1e-6
_LN_EPS = 1e-5


def _slab_kernel(bi, m_ref, x_full_ref, w_in_ref, b_in_ref, xi_ref,
                 w_out_ref, b_out_ref, gamma_ref, beta_ref, out_ref, t_ref):
    i = pl.program_id(0)

    @pl.when(i == 0)
    def _():
        n = x_full_ref.shape[0]
        t_ref[pl.ds(0, n), :] = (
            jax.lax.dot_general(
                x_full_ref[...], w_in_ref[...], (((1,), (1,)), ((), ())),
                preferred_element_type=jnp.float32,
            )
            + b_in_ref[...]
        )

    m = m_ref[...]
    c = jnp.sum(m, axis=0, keepdims=True)
    s = jnp.where(m > _THRESHOLD * (c + _EPS), m, 0.0)
    wsum = jnp.sum(s, axis=0, keepdims=True)
    t = t_ref[pl.ds(0, m.shape[0]), :]
    ws_t = jax.lax.dot_general(
        t, s, (((0,), (0,)), ((), ())),
        preferred_element_type=jnp.float32,
    )
    inv = 1.0 / (c + _EPS)
    factor = inv / (wsum * inv + _EPS)
    has = wsum > 0.0
    ti_t = jnp.transpose(t_ref[pl.ds(i * bi, bi), :])
    agg_t = jnp.where(has, ws_t * factor, ti_t)
    o_t = (
        jax.lax.dot_general(
            w_out_ref[...], agg_t, (((1,), (0,)), ((), ())),
            preferred_element_type=jnp.float32,
        )
        + jnp.transpose(b_out_ref[...])
        + jnp.transpose(xi_ref[...])
    )
    mu = jnp.mean(o_t, axis=0, keepdims=True)
    var = jnp.mean((o_t - mu) ** 2, axis=0, keepdims=True)
    n_t = (o_t - mu) * jax.lax.rsqrt(var + _LN_EPS)
    out_ref[...] = jnp.transpose(
        n_t * jnp.transpose(gamma_ref[...]) + jnp.transpose(beta_ref[...])
    )


@jax.jit
def kernel(node_features, mobility_matrix, W_in, b_in, W_out, b_out, gamma, beta):
    n, d_in = node_features.shape
    d_out = W_in.shape[0]

    bi = 512
    ni = pl.cdiv(n, bi)

    b_in2 = b_in.reshape(1, d_out)
    b_out2 = b_out.reshape(1, d_out)
    gamma2 = gamma.reshape(1, d_out)
    beta2 = beta.reshape(1, d_out)

    out = pl.pallas_call(
        functools.partial(_slab_kernel, bi),
        grid=(ni,),
        in_specs=[
            pl.BlockSpec((n, bi), lambda i: (0, i)),
            pl.BlockSpec((n, d_in), lambda i: (0, 0)),
            pl.BlockSpec((d_out, d_in), lambda i: (0, 0)),
            pl.BlockSpec((1, d_out), lambda i: (0, 0)),
            pl.BlockSpec((bi, d_in), lambda i: (i, 0)),
            pl.BlockSpec((d_out, d_out), lambda i: (0, 0)),
            pl.BlockSpec((1, d_out), lambda i: (0, 0)),
            pl.BlockSpec((1, d_out), lambda i: (0, 0)),
            pl.BlockSpec((1, d_out), lambda i: (0, 0)),
        ],
        out_specs=pl.BlockSpec((bi, d_out), lambda i: (i, 0)),
        out_shape=jax.ShapeDtypeStruct((n, d_out), jnp.float32),
        scratch_shapes=[
            pltpu.VMEM((ni * bi, d_out), jnp.float32),
        ],
        compiler_params=pltpu.CompilerParams(
            dimension_semantics=("arbitrary",),
        ),
    )(mobility_matrix, node_features, W_in, b_in2, node_features,
      W_out, b_out2, gamma2, beta2)

    return out


# speedup vs baseline: 1.0203x; 1.0203x over previous
import functools

import jax
import jax.numpy as jnp
from jax.experimental import pallas as pl
from jax.experimental.pallas import tpu as pltpu

_EPS = 1e-8
_THRESHOLD = 1e-6
_LN_EPS = 1e-5


def _slab_kernel(bi, m_ref, x_full_ref, w_in_ref, b_in_ref, xi_ref,
                 w_out_ref, b_out_ref, gamma_ref, beta_ref, out_ref, taug_ref):
    i = pl.program_id(0)
    n = x_full_ref.shape[0]
    d = w_in_ref.shape[0]

    @pl.when(i == 0)
    def _():
        t = (
            jax.lax.dot_general(
                x_full_ref[...], w_in_ref[...], (((1,), (1,)), ((), ())),
                preferred_element_type=jnp.float32,
            )
            + b_in_ref[...]
        )
        taug_ref[pl.ds(0, d), pl.ds(0, n)] = jnp.transpose(t)
        taug_ref[pl.ds(d, 8), :] = jnp.ones((8, taug_ref.shape[1]), jnp.float32)

    m = m_ref[...]
    c = jnp.sum(m, axis=0, keepdims=True)
    s = jnp.where(m > _THRESHOLD * (c + _EPS), m, 0.0)
    ws_aug = jax.lax.dot_general(
        taug_ref[:, pl.ds(0, n)], s, (((1,), (0,)), ((), ())),
        preferred_element_type=jnp.float32,
    )
    ws_t = ws_aug[0:d, :]
    wsum = ws_aug[d:d + 1, :]
    inv = 1.0 / (c + _EPS)
    factor = inv / (wsum * inv + _EPS)
    has = wsum > 0.0
    ti_t = taug_ref[pl.ds(0, d), pl.ds(i * bi, bi)]
    agg_t = jnp.where(has, ws_t * factor, ti_t)
    o_t = (
        jax.lax.dot_general(
            w_out_ref[...], agg_t, (((1,), (0,)), ((), ())),
            preferred_element_type=jnp.float32,
        )
        + jnp.transpose(b_out_ref[...])
        + jnp.transpose(xi_ref[...])
    )
    mu = jnp.mean(o_t, axis=0, keepdims=True)
    var = jnp.mean((o_t - mu) ** 2, axis=0, keepdims=True)
    n_t = (o_t - mu) * jax.lax.rsqrt(var + _LN_EPS)
    out_ref[...] = jnp.transpose(
        n_t * jnp.transpose(gamma_ref[...]) + jnp.transpose(beta_ref[...])
    )


@jax.jit
def kernel(node_features, mobility_matrix, W_in, b_in, W_out, b_out, gamma, beta):
    n, d_in = node_features.shape
    d_out = W_in.shape[0]

    bi = 512
    ni = pl.cdiv(n, bi)

    b_in2 = b_in.reshape(1, d_out)
    b_out2 = b_out.reshape(1, d_out)
    gamma2 = gamma.reshape(1, d_out)
    beta2 = beta.reshape(1, d_out)

    out = pl.pallas_call(
        functools.partial(_slab_kernel, bi),
        grid=(ni,),
        in_specs=[
            pl.BlockSpec((n, bi), lambda i: (0, i)),
            pl.BlockSpec((n, d_in), lambda i: (0, 0)),
            pl.BlockSpec((d_out, d_in), lambda i: (0, 0)),
            pl.BlockSpec((1, d_out), lambda i: (0, 0)),
            pl.BlockSpec((bi, d_in), lambda i: (i, 0)),
            pl.BlockSpec((d_out, d_out), lambda i: (0, 0)),
            pl.BlockSpec((1, d_out), lambda i: (0, 0)),
            pl.BlockSpec((1, d_out), lambda i: (0, 0)),
            pl.BlockSpec((1, d_out), lambda i: (0, 0)),
        ],
        out_specs=pl.BlockSpec((bi, d_out), lambda i: (i, 0)),
        out_shape=jax.ShapeDtypeStruct((n, d_out), jnp.float32),
        scratch_shapes=[
            pltpu.VMEM((d_out + 8, ni * bi), jnp.float32),
        ],
        compiler_params=pltpu.CompilerParams(
            dimension_semantics=("arbitrary",),
        ),
    )(mobility_matrix, node_features, W_in, b_in2, node_features,
      W_out, b_out2, gamma2, beta2)

    return out
